# Initial kernel scaffold; baseline (speedup 1.0000x reference)
import jax
import jax.numpy as jnp
from jax.experimental import pallas as pl


def kernel(x, tables):
    raise NotImplementedError("write your pallas kernel here")


# speedup vs baseline: 1.0759x; 1.0759x over previous
import functools

import jax
import jax.numpy as jnp
from jax import lax
from jax.experimental import pallas as pl
from jax.experimental.pallas import tpu as pltpu
from jax.experimental.pallas import tpu_sc as plsc

FEATURE_NUM = 26
VOCAB = 100000
EMBED_DIM = 32
BATCH = 16384

_L = 16
_NC = 2
_NS = 16
_NW = _NC * _NS

_ROWS = BATCH * FEATURE_NUM
_CHUNK = 128
_ROWS_PER_W = _ROWS // _NW
_NCHUNK = _ROWS_PER_W // _CHUNK


def _body(x_hbm, table_hbm, out_hbm, idx_v, rows_v, gsem):
    wid = lax.axis_index("s") * _NC + lax.axis_index("c")
    base_chunk = wid * _NCHUNK
    base_row = wid * _ROWS_PER_W

    pltpu.sync_copy(x_hbm.at[pl.ds(base_chunk, _NCHUNK)], idx_v)

    lanes = lax.iota(jnp.int32, _L)

    def compute_chunk(j, _):
        for t in range(_CHUNK // _L):
            r0 = base_row + j * _CHUNK + t * _L
            r = r0 + lanes
            f = lax.rem(r, FEATURE_NUM)
            idx_v[j, pl.ds(t * _L, _L)] = (
                idx_v[j, pl.ds(t * _L, _L)] + f * VOCAB
            )
        return 0

    lax.fori_loop(0, _NCHUNK, compute_chunk, 0)

    def gather_chunk(j, _):
        pltpu.async_copy(table_hbm.at[idx_v.at[j]], rows_v, gsem).wait()
        pltpu.sync_copy(rows_v, out_hbm.at[pl.ds(base_row + j * _CHUNK, _CHUNK)])
        return 0

    lax.fori_loop(0, _NCHUNK, gather_chunk, 0)


@jax.jit
def _run(x2d, table):
    kfn = pl.kernel(
        _body,
        mesh=plsc.VectorSubcoreMesh(core_axis_name="c", subcore_axis_name="s"),
        out_type=jax.ShapeDtypeStruct((_ROWS, EMBED_DIM), jnp.float32),
        scratch_types=[
            pltpu.VMEM((_NCHUNK, _CHUNK), jnp.int32),
            pltpu.VMEM((_CHUNK, EMBED_DIM), jnp.float32),
            pltpu.SemaphoreType.DMA,
        ],
        compiler_params=pltpu.CompilerParams(use_tc_tiling_on_sc=False),
    )
    return kfn(x2d, table)


def kernel(x, tables):
    x2d = x.astype(jnp.int32).reshape(_ROWS // _CHUNK, _CHUNK)
    table = tables.reshape(FEATURE_NUM * VOCAB, EMBED_DIM)
    out = _run(x2d, table)
    return out.reshape(BATCH, FEATURE_NUM, EMBED_DIM)


# speedup vs baseline: 1.1327x; 1.0528x over previous
import functools

import jax
import jax.numpy as jnp
from jax import lax
from jax.experimental import pallas as pl
from jax.experimental.pallas import tpu as pltpu
from jax.experimental.pallas import tpu_sc as plsc

FEATURE_NUM = 26
VOCAB = 100000
EMBED_DIM = 32
BATCH = 16384

_L = 16
_NC = 2
_NS = 16
_NW = _NC * _NS

_ROWS = BATCH * FEATURE_NUM
_CHUNK = 128
_ROWS_PER_W = _ROWS // _NW
_NCHUNK = _ROWS_PER_W // _CHUNK
_CPS = 8
_SUPER = _CPS * _CHUNK
_NSUPER = _ROWS_PER_W // _SUPER


def _body(x_hbm, table_hbm, out_hbm, idx_v, rows_v, gsem, wsem):
    wid = lax.axis_index("s") * _NC + lax.axis_index("c")
    base_chunk = wid * _NCHUNK
    base_row = wid * _ROWS_PER_W

    pltpu.sync_copy(x_hbm.at[pl.ds(base_chunk, _NCHUNK)], idx_v)

    lanes = lax.iota(jnp.int32, _L)

    def compute_chunk(j, _):
        for t in range(_CHUNK // _L):
            r = base_row + j * _CHUNK + t * _L + lanes
            f = lax.rem(r, FEATURE_NUM)
            idx_v[j, pl.ds(t * _L, _L)] = (
                idx_v[j, pl.ds(t * _L, _L)] + f * VOCAB
            )
        return 0

    lax.fori_loop(0, _NCHUNK, compute_chunk, 0)

    def fire_super(ss, buf):
        for k in range(_CPS):
            pltpu.make_async_copy(
                table_hbm.at[idx_v.at[ss * _CPS + k]],
                rows_v.at[buf, pl.ds(k * _CHUNK, _CHUNK)],
                gsem,
            ).start()

    fire_super(0, 0)

    def step(ss, _):
        s = lax.rem(ss, 2)
        s2 = lax.rem(ss + 1, 2)

        @pl.when(ss >= 1)
        def _wait_prev_writeback():
            pltpu.make_async_copy(
                rows_v.at[s2], out_hbm.at[pl.ds(base_row, _SUPER)], wsem
            ).wait()

        @pl.when(ss + 1 < _NSUPER)
        def _fire_next():
            fire_super(ss + 1, s2)

        for k in range(_CPS):
            pltpu.make_async_copy(
                table_hbm.at[idx_v.at[ss * _CPS + k]],
                rows_v.at[s, pl.ds(k * _CHUNK, _CHUNK)],
                gsem,
            ).wait()

        pltpu.make_async_copy(
            rows_v.at[s], out_hbm.at[pl.ds(base_row + ss * _SUPER, _SUPER)], wsem
        ).start()
        return 0

    lax.fori_loop(0, _NSUPER, step, 0)

    pltpu.make_async_copy(
        rows_v.at[(_NSUPER - 1) % 2],
        out_hbm.at[pl.ds(base_row, _SUPER)],
        wsem,
    ).wait()


@jax.jit
def _run(x2d, table):
    kfn = pl.kernel(
        _body,
        mesh=plsc.VectorSubcoreMesh(core_axis_name="c", subcore_axis_name="s"),
        out_type=jax.ShapeDtypeStruct((_ROWS, EMBED_DIM), jnp.float32),
        scratch_types=[
            pltpu.VMEM((_NCHUNK, _CHUNK), jnp.int32),
            pltpu.VMEM((2, _SUPER, EMBED_DIM), jnp.float32),
            pltpu.SemaphoreType.DMA,
            pltpu.SemaphoreType.DMA,
        ],
        compiler_params=pltpu.CompilerParams(use_tc_tiling_on_sc=False),
    )
    return kfn(x2d, table)


def kernel(x, tables):
    x2d = x.astype(jnp.int32).reshape(_ROWS // _CHUNK, _CHUNK)
    table = tables.reshape(FEATURE_NUM * VOCAB, EMBED_DIM)
    out = _run(x2d, table)
    return out.reshape(BATCH, FEATURE_NUM, EMBED_DIM)
